# baseline (speedup 1.0000x reference)
import jax
import jax.numpy as jnp
from jax import lax
from jax.experimental import pallas as pl
from jax.experimental.pallas import tpu as pltpu
from jax.experimental.pallas import tpu_sc as plsc

_CH = 12
_WS = 160
_XYZ_MIN = -1.0
_XYZ_MAX = 1.0
_ROW = 128
_NVE = _WS * _WS * _WS * _CH
_VR = _NVE // _ROW

_NC = 2
_NS = 16
_NW = _NC * _NS

_P = 32
_G = _P // 16
_RPC = 8 * _P

_COFF = (0, _WS, _WS * _WS, _WS * _WS + _WS)


def _make_kernel(npts):
    ppw = npts // _NW
    nchunk = ppw // _P
    assert ppw * _NW == npts and nchunk * _P == ppw

    mesh = plsc.VectorSubcoreMesh(core_axis_name="c", subcore_axis_name="s",
                                  num_cores=_NC, num_subcores=_NS)

    def body(xyz_hbm, tab_hbm, out_hbm,
             xy_v, fx_v, fy_v, fz_v, e0_v, idx_v, rows_v, outb_v,
             xsem0, xsem1, gsem0, gsem1, osem0, osem1):
        xsem = (xsem0, xsem1)
        gsem = (gsem0, gsem1)
        osem = (osem0, osem1)
        wid = lax.axis_index("s") * _NC + lax.axis_index("c")
        base0 = wid * ppw
        iota = lax.iota(jnp.int32, 16)
        half = jnp.float32(1.0 / (_XYZ_MAX - _XYZ_MIN))
        scale = jnp.float32(_WS - 1)

        def fire_xyz(i, s):
            @pl.when(i < nchunk)
            def _():
                gb = base0 + i * _P
                pltpu.async_copy(xyz_hbm.at[pl.ds(gb, _P), :],
                                 xy_v.at[pl.ds(s * _P, _P), :], xsem[s])

        def prep(i, s):
            @pl.when(i < nchunk)
            def _():
                pltpu.make_async_copy(xyz_hbm.at[pl.ds(0, _P), :],
                                      xy_v.at[pl.ds(s * _P, _P), :],
                                      xsem[s]).wait()

                def grp(g, carry):
                    p0 = g * 16
                    pv = s * _P + p0 + iota
                    xv = plsc.load_gather(xy_v, [pv, jnp.full((16,), 0, jnp.int32)])
                    yv = plsc.load_gather(xy_v, [pv, jnp.full((16,), 1, jnp.int32)])
                    zv = plsc.load_gather(xy_v, [pv, jnp.full((16,), 2, jnp.int32)])
                    px = (xv - _XYZ_MIN) * half * scale
                    py = (yv - _XYZ_MIN) * half * scale
                    pz = (zv - _XYZ_MIN) * half * scale
                    xi = jnp.clip(px.astype(jnp.int32), 0, _WS - 2)
                    yi = jnp.clip(py.astype(jnp.int32), 0, _WS - 2)
                    zi = jnp.clip(pz.astype(jnp.int32), 0, _WS - 2)
                    q = s * _P + p0
                    fx_v[pl.ds(q, 16)] = px - xi.astype(jnp.float32)
                    fy_v[pl.ds(q, 16)] = py - yi.astype(jnp.float32)
                    fz_v[pl.ds(q, 16)] = pz - zi.astype(jnp.float32)
                    base = xi * (_WS * _WS) + yi * _WS + zi
                    for k in range(4):
                        f0 = (base + _COFF[k]) * _CH
                        r0 = lax.shift_right_logical(f0, 7)
                        r1 = jnp.minimum(r0 + 1, _VR - 1)
                        e0 = jnp.bitwise_and(f0, 127)
                        qi = s * _RPC + (2 * k) * _P + p0
                        idx_v[pl.ds(qi, 16)] = r0
                        idx_v[pl.ds(qi + _P, 16)] = r1
                        e0_v[pl.ds((s * 4 + k) * _P + p0, 16)] = e0
                    return carry

                lax.fori_loop(0, _G, grp, None)

                pltpu.async_copy(
                    tab_hbm.at[idx_v.at[pl.ds(s * _RPC, _RPC)]],
                    rows_v.at[pl.ds(s * _RPC, _RPC), :],
                    gsem[s])

        def blendout(i, s):
            gb = base0 + i * _P
            pltpu.make_async_copy(
                tab_hbm.at[idx_v.at[pl.ds(s * _RPC, _RPC)]],
                rows_v.at[pl.ds(s * _RPC, _RPC), :],
                gsem[s]).wait()
            @pl.when(i >= 2)
            def _():
                pltpu.make_async_copy(outb_v.at[pl.ds(s * _P, _P), :],
                                      out_hbm.at[pl.ds(0, _P), :],
                                      osem[s]).wait()

            def grp(g, carry):
                p0 = g * 16
                q = s * _P + p0
                fx = fx_v[pl.ds(q, 16)]
                fy = fy_v[pl.ds(q, 16)]
                fz = fz_v[pl.ds(q, 16)]
                one = jnp.float32(1.0)
                gx = one - fx
                gy = one - fy
                gz = one - fz
                wk = (gx * gy, gx * fy, fx * gy, fx * fy)
                w = [(wkv * gz, wkv * fz) for wkv in wk]
                pv = p0 + iota
                acc = [None] * _CH
                for k in range(4):
                    e0 = e0_v[pl.ds((s * 4 + k) * _P + p0, 16)]
                    rkb = s * _RPC + (2 * k) * _P + pv
                    for dz in range(2):
                        wv = w[k][dz]
                        for c in range(_CH):
                            e = e0 + (dz * _CH + c)
                            row = rkb + lax.shift_right_logical(
                                jnp.bitwise_and(e, 128), 2)
                            col = jnp.bitwise_and(e, 127)
                            val = plsc.load_gather(rows_v, [row, col])
                            if acc[c] is None:
                                acc[c] = wv * val
                            else:
                                acc[c] = acc[c] + wv * val
                ov = s * _P + pv
                for c in range(_CH):
                    plsc.store_scatter(outb_v, [ov, jnp.full((16,), c, jnp.int32)],
                                       acc[c])
                return carry

            lax.fori_loop(0, _G, grp, None)
            pltpu.async_copy(outb_v.at[pl.ds(s * _P, _P), :],
                             out_hbm.at[pl.ds(gb, _P), :], osem[s])

        fire_xyz(0, 0)
        fire_xyz(1, 1)
        prep(0, 0)
        fire_xyz(2, 0)

        def step(i, s):
            prep(i + 1, s ^ 1)
            fire_xyz(i + 3, s ^ 1)
            blendout(i, s)

        def dbl(j, carry):
            i = j * 2
            step(i, 0)
            step(i + 1, 1)
            return carry

        lax.fori_loop(0, nchunk // 2, dbl, None)

        for s in (0, 1):
            pltpu.make_async_copy(outb_v.at[pl.ds(s * _P, _P), :],
                                  out_hbm.at[pl.ds(0, _P), :], osem[s]).wait()

    scratch = [
        pltpu.VMEM((2 * _P, 3), jnp.float32),
        pltpu.VMEM((2 * _P,), jnp.float32),
        pltpu.VMEM((2 * _P,), jnp.float32),
        pltpu.VMEM((2 * _P,), jnp.float32),
        pltpu.VMEM((2 * 4 * _P,), jnp.int32),
        pltpu.VMEM((2 * _RPC,), jnp.int32),
        pltpu.VMEM((2 * _RPC, _ROW), jnp.float32),
        pltpu.VMEM((2 * _P, _CH), jnp.float32),
        pltpu.SemaphoreType.DMA,
        pltpu.SemaphoreType.DMA,
        pltpu.SemaphoreType.DMA,
        pltpu.SemaphoreType.DMA,
        pltpu.SemaphoreType.DMA,
        pltpu.SemaphoreType.DMA,
    ]
    return pl.kernel(body,
                     out_type=jax.ShapeDtypeStruct((npts, _CH), jnp.float32),
                     mesh=mesh,
                     compiler_params=pltpu.CompilerParams(
                         needs_layout_passes=False),
                     scratch_types=scratch)


def kernel(xyz, grid):
    npts = xyz.shape[0]
    one = lax.optimization_barrier(jnp.float32(1.0))
    tab = grid.reshape(_VR, _ROW) * one
    return _make_kernel(npts)(xyz, tab)


# speedup vs baseline: 1.1895x; 1.1895x over previous
import jax
import jax.numpy as jnp
from jax import lax
from jax.experimental import pallas as pl
from jax.experimental.pallas import tpu as pltpu
from jax.experimental.pallas import tpu_sc as plsc

_CH = 12
_WS = 160
_XYZ_MIN = -1.0
_XYZ_MAX = 1.0
_ROW = 128
_NCELL = _WS * _WS * _WS

_NC = 2
_NS = 16
_NW = _NC * _NS

_P = 32
_G = _P // 16
_RPC = 8 * _P

_COFF = (0, _WS, _WS * _WS, _WS * _WS + _WS)


def _make_kernel(npts):
    ppw = npts // _NW
    nchunk = ppw // _P
    assert ppw * _NW == npts and nchunk * _P == ppw

    mesh = plsc.VectorSubcoreMesh(core_axis_name="c", subcore_axis_name="s",
                                  num_cores=_NC, num_subcores=_NS)

    def body(xyz_hbm, tab_hbm, out_hbm,
             xy_v, fx_v, fy_v, fz_v, idx_v, rows_v, outb_v,
             xsem0, xsem1, gsem0, gsem1, osem0, osem1):
        xsem = (xsem0, xsem1)
        gsem = (gsem0, gsem1)
        osem = (osem0, osem1)
        wid = lax.axis_index("s") * _NC + lax.axis_index("c")
        base0 = wid * ppw
        iota = lax.iota(jnp.int32, 16)
        half = jnp.float32(1.0 / (_XYZ_MAX - _XYZ_MIN))
        scale = jnp.float32(_WS - 1)

        def fire_xyz(i, s):
            @pl.when(i < nchunk)
            def _():
                gb = base0 + i * _P
                pltpu.async_copy(xyz_hbm.at[pl.ds(gb, _P), :],
                                 xy_v.at[pl.ds(s * _P, _P), :], xsem[s])

        def prep(i, s):
            @pl.when(i < nchunk)
            def _():
                pltpu.make_async_copy(xyz_hbm.at[pl.ds(0, _P), :],
                                      xy_v.at[pl.ds(s * _P, _P), :],
                                      xsem[s]).wait()

                def grp(g, carry):
                    p0 = g * 16
                    pv = s * _P + p0 + iota
                    xv = plsc.load_gather(xy_v, [pv, jnp.full((16,), 0, jnp.int32)])
                    yv = plsc.load_gather(xy_v, [pv, jnp.full((16,), 1, jnp.int32)])
                    zv = plsc.load_gather(xy_v, [pv, jnp.full((16,), 2, jnp.int32)])
                    px = (xv - _XYZ_MIN) * half * scale
                    py = (yv - _XYZ_MIN) * half * scale
                    pz = (zv - _XYZ_MIN) * half * scale
                    xi = jnp.clip(px.astype(jnp.int32), 0, _WS - 2)
                    yi = jnp.clip(py.astype(jnp.int32), 0, _WS - 2)
                    zi = jnp.clip(pz.astype(jnp.int32), 0, _WS - 2)
                    q = s * _P + p0
                    fx_v[pl.ds(q, 16)] = px - xi.astype(jnp.float32)
                    fy_v[pl.ds(q, 16)] = py - yi.astype(jnp.float32)
                    fz_v[pl.ds(q, 16)] = pz - zi.astype(jnp.float32)
                    base = xi * (_WS * _WS) + yi * _WS + zi
                    for k in range(4):
                        r0 = base + _COFF[k]
                        qi = s * _RPC + (2 * k) * _P + p0
                        idx_v[pl.ds(qi, 16)] = r0
                        idx_v[pl.ds(qi + _P, 16)] = r0 + 1
                    return carry

                lax.fori_loop(0, _G, grp, None)

                pltpu.async_copy(
                    tab_hbm.at[idx_v.at[pl.ds(s * _RPC, _RPC)]],
                    rows_v.at[pl.ds(s * _RPC, _RPC), :],
                    gsem[s])

        def blendout(i, s):
            gb = base0 + i * _P
            pltpu.make_async_copy(
                tab_hbm.at[idx_v.at[pl.ds(s * _RPC, _RPC)]],
                rows_v.at[pl.ds(s * _RPC, _RPC), :],
                gsem[s]).wait()
            @pl.when(i >= 2)
            def _():
                pltpu.make_async_copy(outb_v.at[pl.ds(s * _P, _P), :],
                                      out_hbm.at[pl.ds(0, _P), :],
                                      osem[s]).wait()

            def grp(g, carry):
                p0 = g * 16
                q = s * _P + p0
                fx = fx_v[pl.ds(q, 16)]
                fy = fy_v[pl.ds(q, 16)]
                fz = fz_v[pl.ds(q, 16)]
                one = jnp.float32(1.0)
                gx = one - fx
                gy = one - fy
                gz = one - fz
                wk = (gx * gy, gx * fy, fx * gy, fx * fy)
                w = [(wkv * gz, wkv * fz) for wkv in wk]
                pv = p0 + iota
                acc = [None] * _CH
                for k in range(4):
                    for dz in range(2):
                        row = s * _RPC + (2 * k + dz) * _P + pv
                        wv = w[k][dz]
                        for c in range(_CH):
                            val = plsc.load_gather(
                                rows_v, [row, jnp.full((16,), c, jnp.int32)])
                            if acc[c] is None:
                                acc[c] = wv * val
                            else:
                                acc[c] = acc[c] + wv * val
                ov = s * _P + pv
                for c in range(_CH):
                    plsc.store_scatter(outb_v, [ov, jnp.full((16,), c, jnp.int32)],
                                       acc[c])
                return carry

            lax.fori_loop(0, _G, grp, None)
            pltpu.async_copy(outb_v.at[pl.ds(s * _P, _P), :],
                             out_hbm.at[pl.ds(gb, _P), :], osem[s])

        fire_xyz(0, 0)
        fire_xyz(1, 1)
        prep(0, 0)
        fire_xyz(2, 0)

        def step(i, s):
            prep(i + 1, s ^ 1)
            fire_xyz(i + 3, s ^ 1)
            blendout(i, s)

        def dbl(j, carry):
            i = j * 2
            step(i, 0)
            step(i + 1, 1)
            return carry

        lax.fori_loop(0, nchunk // 2, dbl, None)

        for s in (0, 1):
            pltpu.make_async_copy(outb_v.at[pl.ds(s * _P, _P), :],
                                  out_hbm.at[pl.ds(0, _P), :], osem[s]).wait()

    scratch = [
        pltpu.VMEM((2 * _P, 3), jnp.float32),
        pltpu.VMEM((2 * _P,), jnp.float32),
        pltpu.VMEM((2 * _P,), jnp.float32),
        pltpu.VMEM((2 * _P,), jnp.float32),
        pltpu.VMEM((2 * _RPC,), jnp.int32),
        pltpu.VMEM((2 * _RPC, _ROW), jnp.float32),
        pltpu.VMEM((2 * _P, _CH), jnp.float32),
        pltpu.SemaphoreType.DMA,
        pltpu.SemaphoreType.DMA,
        pltpu.SemaphoreType.DMA,
        pltpu.SemaphoreType.DMA,
        pltpu.SemaphoreType.DMA,
        pltpu.SemaphoreType.DMA,
    ]
    return pl.kernel(body,
                     out_type=jax.ShapeDtypeStruct((npts, _CH), jnp.float32),
                     mesh=mesh,
                     compiler_params=pltpu.CompilerParams(
                         needs_layout_passes=False),
                     scratch_types=scratch)


def kernel(xyz, grid):
    npts = xyz.shape[0]
    tab = jnp.pad(grid.reshape(_NCELL, _CH), ((0, 0), (0, _ROW - _CH)))
    return _make_kernel(npts)(xyz, tab)
